# baseline (speedup 1.0000x reference)
import functools

import jax
import jax.numpy as jnp
from jax import lax
from jax.experimental import pallas as pl
from jax.experimental.pallas import tpu as pltpu
from jax.experimental.pallas import tpu_sc as plsc

N_NODES = 10000
HIDDEN = 128
NUM_LAYERS = 3
BATCH = 256

_NC, _NS = 2, 16
_NW = _NC * _NS
_EC = 128
_CHUNKS = 80
_EPAD = _NW * _CHUNKS * _EC
_NPAD = 10240
_RPT = _NPAD // _NS
_GC = 3
_GPAD = _NW * _GC * _EC
_BPW = BATCH // _NW
_HALF = 5120
_CH2 = 160
_ZROWS = 2288
_HMP = 12288
_RPT2 = _HALF // _NS

_MESH = plsc.VectorSubcoreMesh(core_axis_name="c", subcore_axis_name="s")


def _f32(*shape):
    return jax.ShapeDtypeStruct(shape, jnp.float32)


def _sc_emb_gather(table, idx2d):
    @functools.partial(
        pl.kernel,
        out_type=_f32(_GPAD, HIDDEN),
        mesh=_MESH,
        scratch_types=[
            pltpu.VMEM((_GC, _EC), jnp.int32),
            pltpu.VMEM((_GC * _EC, HIDDEN), jnp.float32),
            pltpu.SemaphoreType.DMA,
        ],
    )
    def k(tab_hbm, idx_hbm, out_hbm, idx_v, rows_v, sem):
        w = lax.axis_index("s") * _NC + lax.axis_index("c")
        pltpu.sync_copy(idx_hbm.at[w], idx_v)
        for j in range(_GC):
            pltpu.async_copy(tab_hbm.at[idx_v.at[j]],
                             rows_v.at[pl.ds(j * _EC, _EC)], sem)
        for j in range(_GC):
            pltpu.make_async_copy(tab_hbm.at[idx_v.at[j]],
                                  rows_v.at[pl.ds(j * _EC, _EC)], sem).wait()
        pltpu.sync_copy(rows_v, out_hbm.at[pl.ds(w * _GC * _EC, _GC * _EC)])

    return k(table, idx2d)


def _sc_edge_agg(hmp, srcb, dstb, zstripe):
    @functools.partial(
        pl.kernel,
        out_type=_f32(2 * _HALF, HIDDEN),
        mesh=_MESH,
        scratch_types=[
            pltpu.VMEM((_CH2, _EC), jnp.int32),
            pltpu.VMEM((_CH2, _EC), jnp.int32),
            pltpu.VMEM((_EC, HIDDEN), jnp.float32),
            pltpu.VMEM((_EC, HIDDEN), jnp.float32),
            pltpu.VMEM((_EC, HIDDEN), jnp.float32),
            pltpu.VMEM((_EC, HIDDEN), jnp.float32),
            pltpu.VMEM_SHARED((_HALF, HIDDEN), jnp.float32),
            pltpu.SemaphoreType.DMA,
            pltpu.SemaphoreType.DMA,
        ],
    )
    def k(hm_hbm, src_hbm, dst_hbm, z_hbm, out_hbm, src_v, dst_v,
          b0, b1, b2, b3, agg_sh, gsem, ssem):
        bufs = [b0, b1, b2, b3]
        c = lax.axis_index("c")
        s = lax.axis_index("s")
        base = c * (2 * _CH2 * _NS // 2) + s * _CH2
        pltpu.sync_copy(z_hbm, agg_sh.at[pl.ds(s * _RPT2, _RPT2)])
        pltpu.sync_copy(src_hbm.at[pl.ds(base, _CH2)], src_v)
        pltpu.sync_copy(dst_hbm.at[pl.ds(base, _CH2)], dst_v)
        plsc.subcore_barrier()

        def issue_g(j, bi):
            pltpu.async_copy(hm_hbm.at[src_v.at[j]], bufs[bi], gsem)

        def wait_g(j, bi):
            pltpu.make_async_copy(hm_hbm.at[src_v.at[j]], bufs[bi],
                                  gsem).wait()

        def issue_s(j, bi):
            pltpu.async_copy(bufs[bi], agg_sh.at[dst_v.at[j]], ssem,
                             add=True)

        def wait_s(j, bi):
            pltpu.make_async_copy(bufs[bi], agg_sh.at[dst_v.at[j]],
                                  ssem).wait()

        @pl.loop(0, _CH2, step=2)
        def _(j0):
            for bi in range(2):
                issue_g(j0 + bi, bi)
            for bi in range(2):
                wait_g(j0 + bi, bi)
                issue_s(j0 + bi, bi)
            for bi in range(2):
                wait_s(j0 + bi, bi)

        plsc.subcore_barrier()
        pltpu.sync_copy(agg_sh.at[pl.ds(s * _RPT2, _RPT2)],
                        out_hbm.at[pl.ds(c * _HALF + s * _RPT2, _RPT2)])

    return k(hmp, srcb, dstb, zstripe)


def _sc_root_gather(h, roots):
    @functools.partial(
        pl.kernel,
        out_type=_f32(BATCH, HIDDEN),
        mesh=_MESH,
        scratch_types=[
            pltpu.VMEM((_BPW,), jnp.int32),
            pltpu.VMEM((_BPW, HIDDEN), jnp.float32),
            pltpu.SemaphoreType.DMA,
        ],
    )
    def k(h_hbm, r_hbm, out_hbm, idx_v, rows_v, sem):
        w = lax.axis_index("s") * _NC + lax.axis_index("c")
        pltpu.sync_copy(r_hbm.at[pl.ds(w * _BPW, _BPW)], idx_v)
        pltpu.async_copy(h_hbm.at[idx_v], rows_v, sem).wait()
        pltpu.sync_copy(rows_v, out_hbm.at[pl.ds(w * _BPW, _BPW)])

    return k(h, roots)


_BLK = 1000
_NBLK = N_NODES // _BLK

_row_spec = pl.BlockSpec((_BLK, HIDDEN), lambda i: (i, 0))
_w_spec = pl.BlockSpec((HIDDEN, HIDDEN), lambda i: (0, 0))
_b_spec = pl.BlockSpec((1, HIDDEN), lambda i: (0, 0))


def _tc_mm_first(h0, Ws, Wm, bias):
    def body(h_ref, ws_ref, wm_ref, b_ref, hs_ref, hm_ref):
        hb = h_ref[...]
        hs_ref[...] = (jnp.dot(hb, ws_ref[...],
                               preferred_element_type=jnp.float32)
                       + b_ref[...])
        hm_ref[...] = jnp.dot(hb, wm_ref[...],
                              preferred_element_type=jnp.float32)

    return pl.pallas_call(
        body,
        grid=(_NBLK,),
        in_specs=[_row_spec, _w_spec, _w_spec, _b_spec],
        out_specs=[_row_spec, _row_spec],
        out_shape=[_f32(N_NODES, HIDDEN)] * 2,
    )(h0, Ws, Wm, bias)


def _tc_mm_mid(hs_in, a0, Ws, Wm, bias):
    def body(hs_ref, a0_ref, ws_ref, wm_ref, b_ref, hs_o, hm_o):
        h = jnp.maximum(hs_ref[...] + a0_ref[...], 0.0)
        hs_o[...] = (jnp.dot(h, ws_ref[...],
                             preferred_element_type=jnp.float32)
                     + b_ref[...])
        hm_o[...] = jnp.dot(h, wm_ref[...],
                            preferred_element_type=jnp.float32)

    return pl.pallas_call(
        body,
        grid=(_NBLK,),
        in_specs=[_row_spec, _row_spec, _w_spec, _w_spec, _b_spec],
        out_specs=[_row_spec, _row_spec],
        out_shape=[_f32(N_NODES, HIDDEN)] * 2,
    )(hs_in, a0, Ws, Wm, bias)


def _tc_relu_sum(hs_in, a0):
    def body(hs_ref, a0_ref, h_o):
        h_o[...] = jnp.maximum(hs_ref[...] + a0_ref[...], 0.0)

    return pl.pallas_call(
        body,
        grid=(_NBLK,),
        in_specs=[_row_spec, _row_spec],
        out_specs=_row_spec,
        out_shape=_f32(N_NODES, HIDDEN),
    )(hs_in, a0)


def kernel(node_wids, edge_index, root_idxs, emb_table, W_self, W_msg, b):
    src = edge_index[0]
    dst = edge_index[1]
    pad_e = _EPAD - src.shape[0]
    pos = jnp.arange(_EPAD, dtype=jnp.int32)
    src_p = jnp.concatenate(
        [src, N_NODES + jnp.arange(pad_e, dtype=jnp.int32) % _ZROWS])
    dst_p = jnp.concatenate(
        [dst, jnp.arange(pad_e, dtype=jnp.int32) % _HALF])
    in0 = dst_p < _HALF
    zsrc = N_NODES + pos % _ZROWS
    zdst = pos % _HALF
    srcA = jnp.where(in0, src_p, zsrc)
    dstA = jnp.where(in0, dst_p, zdst)
    srcB = jnp.where(in0, zsrc, src_p)
    dstB = jnp.where(in0, zdst, dst_p - _HALF)
    srcb = jnp.concatenate([srcA, srcB]).reshape(2 * _NS * _CH2, _EC)
    dstb = jnp.concatenate([dstA, dstB]).reshape(2 * _NS * _CH2, _EC)
    wid_pad = jnp.arange(_GPAD - N_NODES, dtype=jnp.int32) % 779
    wid2d = jnp.concatenate([node_wids, wid_pad]).reshape(_NW, _GC, _EC)
    zstripe = jnp.zeros((_RPT2, HIDDEN), jnp.float32)
    zrows = jnp.zeros((_ZROWS, HIDDEN), jnp.float32)
    bias = b.reshape(NUM_LAYERS, 1, HIDDEN)

    h0 = _sc_emb_gather(emb_table, wid2d)[:N_NODES]
    hs, hm = _tc_mm_first(h0, W_self[0], W_msg[0], bias[0])
    h_final = None
    for l in range(NUM_LAYERS):
        hmp = jnp.concatenate([hm, zrows])
        agg = _sc_edge_agg(hmp, srcb, dstb, zstripe)[:N_NODES]
        if l < NUM_LAYERS - 1:
            hs, hm = _tc_mm_mid(hs, agg, W_self[l + 1], W_msg[l + 1],
                                bias[l + 1])
        else:
            h_final = _tc_relu_sum(hs, agg)
    return _sc_root_gather(h_final, root_idxs)


# speedup vs baseline: 1.2826x; 1.2826x over previous
import functools

import jax
import jax.numpy as jnp
from jax import lax
from jax.experimental import pallas as pl
from jax.experimental.pallas import tpu as pltpu
from jax.experimental.pallas import tpu_sc as plsc

N_NODES = 10000
HIDDEN = 128
NUM_LAYERS = 3
BATCH = 256

_NC, _NS = 2, 16
_NW = _NC * _NS
_EC = 128
_CHUNKS = 80
_EPAD = _NW * _CHUNKS * _EC
_NPAD = 10240
_RPT = _NPAD // _NS
_GC = 3
_GPAD = _NW * _GC * _EC
_BPW = BATCH // _NW
_EPW = 10000
_PPW = _CHUNKS * _EC - _EPW

_MESH = plsc.VectorSubcoreMesh(core_axis_name="c", subcore_axis_name="s")


def _f32(*shape):
    return jax.ShapeDtypeStruct(shape, jnp.float32)


def _sc_emb_gather(table, idx2d):
    @functools.partial(
        pl.kernel,
        out_type=_f32(_GPAD, HIDDEN),
        mesh=_MESH,
        scratch_types=[
            pltpu.VMEM((_GC, _EC), jnp.int32),
            pltpu.VMEM((_GC * _EC, HIDDEN), jnp.float32),
            pltpu.SemaphoreType.DMA,
        ],
    )
    def k(tab_hbm, idx_hbm, out_hbm, idx_v, rows_v, sem):
        w = lax.axis_index("s") * _NC + lax.axis_index("c")
        pltpu.sync_copy(idx_hbm.at[w], idx_v)
        for j in range(_GC):
            pltpu.async_copy(tab_hbm.at[idx_v.at[j]],
                             rows_v.at[pl.ds(j * _EC, _EC)], sem)
        for j in range(_GC):
            pltpu.make_async_copy(tab_hbm.at[idx_v.at[j]],
                                  rows_v.at[pl.ds(j * _EC, _EC)], sem).wait()
        pltpu.sync_copy(rows_v, out_hbm.at[pl.ds(w * _GC * _EC, _GC * _EC)])

    return k(table, idx2d)


def _sc_edge_agg(hm, src2d, dst2d, zstripe):
    @functools.partial(
        pl.kernel,
        out_type=_f32(2 * _NPAD, HIDDEN),
        mesh=_MESH,
        scratch_types=[
            pltpu.VMEM((_CHUNKS, _EC), jnp.int32),
            pltpu.VMEM((_CHUNKS, _EC), jnp.int32),
            pltpu.VMEM((_EC, HIDDEN), jnp.float32),
            pltpu.VMEM_SHARED((_NPAD, HIDDEN), jnp.float32),
            pltpu.SemaphoreType.DMA,
        ],
    )
    def k(hm_hbm, src_hbm, dst_hbm, z_hbm, out_hbm, src_v, dst_v, rows_v,
          agg_sh, sem):
        c = lax.axis_index("c")
        s = lax.axis_index("s")
        w = s * _NC + c
        pltpu.sync_copy(z_hbm, agg_sh.at[pl.ds(s * _RPT, _RPT)])
        pltpu.sync_copy(src_hbm.at[pl.ds(w * _CHUNKS, _CHUNKS)], src_v)
        pltpu.sync_copy(dst_hbm.at[pl.ds(w * _CHUNKS, _CHUNKS)], dst_v)
        plsc.subcore_barrier()

        @pl.loop(0, _CHUNKS)
        def _(j):
            pltpu.async_copy(hm_hbm.at[src_v.at[j]], rows_v, sem).wait()
            pltpu.sync_copy(rows_v, agg_sh.at[dst_v.at[j]], add=True)

        plsc.subcore_barrier()
        pltpu.sync_copy(agg_sh.at[pl.ds(s * _RPT, _RPT)],
                        out_hbm.at[pl.ds(c * _NPAD + s * _RPT, _RPT)])

    return k(hm, src2d, dst2d, zstripe)


def _sc_root_gather3(t0, t1, t2, roots):
    @functools.partial(
        pl.kernel,
        out_type=_f32(3 * BATCH, HIDDEN),
        mesh=_MESH,
        scratch_types=[
            pltpu.VMEM((_BPW,), jnp.int32),
            pltpu.VMEM((_BPW, HIDDEN), jnp.float32),
            pltpu.SemaphoreType.DMA,
        ],
    )
    def k(t0_hbm, t1_hbm, t2_hbm, r_hbm, out_hbm, idx_v, rows_v, sem):
        w = lax.axis_index("s") * _NC + lax.axis_index("c")
        pltpu.sync_copy(r_hbm.at[pl.ds(w * _BPW, _BPW)], idx_v)
        for t, tab in enumerate([t0_hbm, t1_hbm, t2_hbm]):
            pltpu.async_copy(tab.at[idx_v], rows_v, sem).wait()
            pltpu.sync_copy(rows_v,
                            out_hbm.at[pl.ds(t * BATCH + w * _BPW, _BPW)])

    return k(t0, t1, t2, roots)


_BLK = 1000
_NBLK = N_NODES // _BLK

_row_spec = pl.BlockSpec((_BLK, HIDDEN), lambda i: (i, 0))
_w_spec = pl.BlockSpec((HIDDEN, HIDDEN), lambda i: (0, 0))
_b_spec = pl.BlockSpec((1, HIDDEN), lambda i: (0, 0))


def _tc_msg(h, Wm):
    def body(h_ref, wm_ref, hm_ref):
        hm_ref[...] = jnp.dot(h_ref[...], wm_ref[...],
                              preferred_element_type=jnp.float32)

    return pl.pallas_call(
        body,
        grid=(_NBLK,),
        in_specs=[_row_spec, _w_spec],
        out_specs=_row_spec,
        out_shape=_f32(N_NODES, HIDDEN),
    )(h, Wm)


def _tc_self(h, Ws, bias):
    def body(h_ref, ws_ref, b_ref, hs_ref):
        hs_ref[...] = (jnp.dot(h_ref[...], ws_ref[...],
                               preferred_element_type=jnp.float32)
                       + b_ref[...])

    return pl.pallas_call(
        body,
        grid=(_NBLK,),
        in_specs=[_row_spec, _w_spec, _b_spec],
        out_specs=_row_spec,
        out_shape=_f32(N_NODES, HIDDEN),
    )(h, Ws, bias)


def _tc_relu_msg(hs, a0, a1, Wm):
    def body(hs_ref, a0_ref, a1_ref, wm_ref, h_ref, hm_ref):
        h = jnp.maximum(hs_ref[...] + a0_ref[...] + a1_ref[...], 0.0)
        h_ref[...] = h
        hm_ref[...] = jnp.dot(h, wm_ref[...],
                              preferred_element_type=jnp.float32)

    return pl.pallas_call(
        body,
        grid=(_NBLK,),
        in_specs=[_row_spec, _row_spec, _row_spec, _w_spec],
        out_specs=[_row_spec, _row_spec],
        out_shape=[_f32(N_NODES, HIDDEN)] * 2,
    )(hs, a0, a1, Wm)


def _tc_root_relu(r3):
    def body(r_ref, o_ref):
        o_ref[...] = jnp.maximum(
            r_ref[pl.ds(0, BATCH), :] + r_ref[pl.ds(BATCH, BATCH), :]
            + r_ref[pl.ds(2 * BATCH, BATCH), :], 0.0)

    return pl.pallas_call(
        body,
        grid=(1,),
        in_specs=[pl.BlockSpec((3 * BATCH, HIDDEN), lambda i: (0, 0))],
        out_specs=pl.BlockSpec((BATCH, HIDDEN), lambda i: (0, 0)),
        out_shape=_f32(BATCH, HIDDEN),
    )(r3)


def kernel(node_wids, edge_index, root_idxs, emb_table, W_self, W_msg, b):
    src = edge_index[0]
    dst = edge_index[1]
    pad_src = jnp.broadcast_to(jnp.arange(_PPW, dtype=jnp.int32)[None, :],
                               (_NW, _PPW))
    pad_dst = jnp.broadcast_to(
        (N_NODES + jnp.arange(_PPW, dtype=jnp.int32))[None, :], (_NW, _PPW))
    src2d = jnp.concatenate([src.reshape(_NW, _EPW), pad_src],
                            axis=1).reshape(_NW * _CHUNKS, _EC)
    dst2d = jnp.concatenate([dst.reshape(_NW, _EPW), pad_dst],
                            axis=1).reshape(_NW * _CHUNKS, _EC)
    wid_pad = jnp.arange(_GPAD - N_NODES, dtype=jnp.int32) % 779
    wid2d = jnp.concatenate([node_wids, wid_pad]).reshape(_NW, _GC, _EC)
    zstripe = jnp.zeros((_RPT, HIDDEN), jnp.float32)
    bias = b.reshape(NUM_LAYERS, 1, HIDDEN)

    h0 = _sc_emb_gather(emb_table, wid2d)[:N_NODES]
    hm = _tc_msg(h0, W_msg[0])
    hs = _tc_self(h0, W_self[0], bias[0])
    for l in range(NUM_LAYERS):
        agg = _sc_edge_agg(hm, src2d, dst2d, zstripe)
        a0 = agg[:N_NODES]
        a1 = agg[_NPAD:_NPAD + N_NODES]
        if l < NUM_LAYERS - 1:
            h, hm = _tc_relu_msg(hs, a0, a1, W_msg[l + 1])
            hs = _tc_self(h, W_self[l + 1], bias[l + 1])
    r3 = _sc_root_gather3(hs, a0, a1, root_idxs)
    return _tc_root_relu(r3)
